# baseline (speedup 1.0000x reference)
import functools

import jax
import jax.numpy as jnp
from jax import lax
from jax.experimental import pallas as pl
from jax.experimental.pallas import tpu as pltpu
from jax.experimental.pallas import tpu_sc as plsc

_EMBED = 32
_CODES = 8192
_BLOCK_M = 128


_LANES = 128


def _dist_argmin_body(xn_ref, wn_ref, x2_ref, w_ref, idx_ref):
    x2 = x2_ref[...]
    w = w_ref[...]
    mm2 = lax.dot_general(
        x2.astype(jnp.bfloat16), w.astype(jnp.bfloat16),
        (((1,), (1,)), ((), ())), preferred_element_type=jnp.float32
    )
    xn = xn_ref[...]
    wn = wn_ref[...]
    nb = _CODES // 2
    ns = nb // _LANES
    lane = lax.broadcasted_iota(jnp.int32, (_BLOCK_M, _LANES), 1)
    acc_q = jnp.full((_BLOCK_M,), jnp.inf, dtype=jnp.float32)
    acc_i = jnp.zeros((_BLOCK_M,), dtype=jnp.int32)
    for b in range(2):
        base = b * nb
        run_v = (xn + wn[:, base:base + _LANES]) - mm2[:, base:base + _LANES]
        run_s = jnp.zeros((_BLOCK_M, _LANES), dtype=jnp.int32)
        for s in range(1, ns):
            lo = base + s * _LANES
            dv = (xn + wn[:, lo:lo + _LANES]) - mm2[:, lo:lo + _LANES]
            lt = dv < run_v
            run_v = jnp.where(lt, dv, run_v)
            run_s = jnp.where(lt, jnp.int32(s), run_s)
        m = jnp.min(run_v, axis=1)
        cand = run_s * jnp.int32(_LANES) + lane
        g = jnp.min(jnp.where(run_v == m[:, None], cand, jnp.int32(_CODES)), axis=1)
        take = m < acc_q
        acc_q = jnp.where(take, m.astype(jnp.bfloat16).astype(jnp.float32), acc_q)
        acc_i = jnp.where(take, g + jnp.int32(base), acc_i)
    idx_ref[...] = acc_i.reshape(1, 1, _BLOCK_M)


def _dist_argmin(xn, wn, flat, w, interpret=False):
    nb = flat.shape[0] // _BLOCK_M
    idx3 = pl.pallas_call(
        _dist_argmin_body,
        grid=(nb,),
        in_specs=[
            pl.BlockSpec((_BLOCK_M, 1), lambda i: (i, 0)),
            pl.BlockSpec((1, _CODES), lambda i: (0, 0)),
            pl.BlockSpec((_BLOCK_M, _EMBED), lambda i: (i, 0)),
            pl.BlockSpec((_CODES, _EMBED), lambda i: (0, 0)),
        ],
        out_specs=pl.BlockSpec((1, 1, _BLOCK_M), lambda i: (i, 0, 0)),
        out_shape=jax.ShapeDtypeStruct((nb, 1, _BLOCK_M), jnp.int32),
        interpret=interpret,
    )(xn, wn, flat, w)
    return idx3.reshape(-1)


_CHUNK = 128


def _sc_gather(table, idx, n_rows):
    info = plsc.get_sparse_core_info()
    nw = info.num_cores * info.num_subcores
    b_per_w = n_rows // nw
    n_ch = b_per_w // _CHUNK
    d = table.shape[1]
    mesh = plsc.VectorSubcoreMesh(core_axis_name="c", subcore_axis_name="s")

    @functools.partial(
        pl.kernel,
        mesh=mesh,
        out_type=jax.ShapeDtypeStruct((n_rows, d), jnp.float32),
        scratch_types=[
            pltpu.VMEM((n_ch, _CHUNK), jnp.int32),
            pltpu.VMEM((b_per_w, d), jnp.float32),
            pltpu.SemaphoreType.DMA,
        ],
        compiler_params=pltpu.CompilerParams(use_tc_tiling_on_sc=False),
    )
    def gather_k(table_hbm, idx_hbm, out_hbm, idx_v, rows_v, sem):
        wid = lax.axis_index("s") * info.num_cores + lax.axis_index("c")
        base = wid * b_per_w
        pltpu.sync_copy(idx_hbm.at[pl.ds(wid * n_ch, n_ch)], idx_v)
        copies = [
            pltpu.async_copy(
                table_hbm.at[idx_v.at[j]],
                rows_v.at[pl.ds(j * _CHUNK, _CHUNK)],
                sem,
            )
            for j in range(n_ch)
        ]
        for c in copies:
            c.wait()
        pltpu.sync_copy(rows_v, out_hbm.at[pl.ds(base, b_per_w)])

    return gather_k(table, idx.reshape(-1, _CHUNK))


def kernel(inputs, W):
    input_shape = inputs.shape
    flat = inputs.reshape(-1, _EMBED)
    n_rows = flat.shape[0]
    xn = jnp.sum(flat ** 2, axis=1, keepdims=True)
    wn = jnp.sum(W ** 2, axis=1).reshape(1, _CODES)
    idx = _dist_argmin(xn, wn, flat * 2.0, W)
    quantized = _sc_gather(W, idx, n_rows).reshape(input_shape)
    quantized_st = inputs + lax.stop_gradient(quantized - inputs)
    indices = idx.reshape(input_shape[:-1])
    return (quantized_st, indices, jnp.array(0.0, dtype=jnp.float32))


# speedup vs baseline: 1.2147x; 1.2147x over previous
import functools

import jax
import jax.numpy as jnp
from jax import lax
from jax.experimental import pallas as pl
from jax.experimental.pallas import tpu as pltpu
from jax.experimental.pallas import tpu_sc as plsc

_EMBED = 32
_CODES = 8192
_BLOCK_M = 512


_LANES = 128


def _dist_argmin_body(xn_ref, wn_ref, x2_ref, w_ref, idx_ref):
    x2 = x2_ref[...]
    w = w_ref[...]
    mm2 = lax.dot_general(
        x2.astype(jnp.bfloat16), w.astype(jnp.bfloat16),
        (((1,), (1,)), ((), ())), preferred_element_type=jnp.float32
    )
    xn = xn_ref[...]
    wn = wn_ref[...]
    nb = _CODES // 2
    ns = nb // _LANES
    lane = lax.broadcasted_iota(jnp.int32, (_BLOCK_M, _LANES), 1)
    acc_q = jnp.full((_BLOCK_M,), jnp.inf, dtype=jnp.float32)
    acc_i = jnp.zeros((_BLOCK_M,), dtype=jnp.int32)
    for b in range(2):
        base = b * nb
        run_v = (xn + wn[:, base:base + _LANES]) - mm2[:, base:base + _LANES]
        run_s = jnp.zeros((_BLOCK_M, _LANES), dtype=jnp.int32)
        for s in range(1, ns):
            lo = base + s * _LANES
            dv = (xn + wn[:, lo:lo + _LANES]) - mm2[:, lo:lo + _LANES]
            lt = dv < run_v
            run_v = jnp.where(lt, dv, run_v)
            run_s = jnp.where(lt, jnp.int32(s), run_s)
        m = jnp.min(run_v, axis=1)
        cand = run_s * jnp.int32(_LANES) + lane
        g = jnp.min(jnp.where(run_v == m[:, None], cand, jnp.int32(_CODES)), axis=1)
        take = m < acc_q
        acc_q = jnp.where(take, m.astype(jnp.bfloat16).astype(jnp.float32), acc_q)
        acc_i = jnp.where(take, g + jnp.int32(base), acc_i)
    idx_ref[...] = acc_i.reshape(1, 1, _BLOCK_M)


def _dist_argmin(xn, wn, flat, w, interpret=False):
    nb = flat.shape[0] // _BLOCK_M
    idx3 = pl.pallas_call(
        _dist_argmin_body,
        grid=(nb,),
        in_specs=[
            pl.BlockSpec((_BLOCK_M, 1), lambda i: (i, 0)),
            pl.BlockSpec((1, _CODES), lambda i: (0, 0)),
            pl.BlockSpec((_BLOCK_M, _EMBED), lambda i: (i, 0)),
            pl.BlockSpec((_CODES, _EMBED), lambda i: (0, 0)),
        ],
        out_specs=pl.BlockSpec((1, 1, _BLOCK_M), lambda i: (i, 0, 0)),
        out_shape=jax.ShapeDtypeStruct((nb, 1, _BLOCK_M), jnp.int32),
        interpret=interpret,
    )(xn, wn, flat, w)
    return idx3.reshape(-1)


_CHUNK = 128


def _sc_gather(table, idx, n_rows):
    info = plsc.get_sparse_core_info()
    nw = info.num_cores * info.num_subcores
    b_per_w = n_rows // nw
    n_ch = b_per_w // _CHUNK
    d = table.shape[1]
    mesh = plsc.VectorSubcoreMesh(core_axis_name="c", subcore_axis_name="s")

    @functools.partial(
        pl.kernel,
        mesh=mesh,
        out_type=jax.ShapeDtypeStruct((n_rows, d), jnp.float32),
        scratch_types=[
            pltpu.VMEM((n_ch, _CHUNK), jnp.int32),
            pltpu.VMEM((b_per_w, d), jnp.float32),
            pltpu.SemaphoreType.DMA,
        ],
        compiler_params=pltpu.CompilerParams(use_tc_tiling_on_sc=False),
    )
    def gather_k(table_hbm, idx_hbm, out_hbm, idx_v, rows_v, sem):
        wid = lax.axis_index("s") * info.num_cores + lax.axis_index("c")
        base = wid * b_per_w
        pltpu.sync_copy(idx_hbm.at[pl.ds(wid * n_ch, n_ch)], idx_v)
        copies = [
            pltpu.async_copy(
                table_hbm.at[idx_v.at[j]],
                rows_v.at[pl.ds(j * _CHUNK, _CHUNK)],
                sem,
            )
            for j in range(n_ch)
        ]
        for c in copies:
            c.wait()
        pltpu.sync_copy(rows_v, out_hbm.at[pl.ds(base, b_per_w)])

    return gather_k(table, idx.reshape(-1, _CHUNK))


def kernel(inputs, W):
    input_shape = inputs.shape
    flat = inputs.reshape(-1, _EMBED)
    n_rows = flat.shape[0]
    xn = jnp.sum(flat ** 2, axis=1, keepdims=True)
    wn = jnp.sum(W ** 2, axis=1).reshape(1, _CODES)
    idx = _dist_argmin(xn, wn, flat * 2.0, W)
    quantized = _sc_gather(W, idx, n_rows).reshape(input_shape)
    quantized_st = inputs + lax.stop_gradient(quantized - inputs)
    indices = idx.reshape(input_shape[:-1])
    return (quantized_st, indices, jnp.array(0.0, dtype=jnp.float32))


# speedup vs baseline: 1.2479x; 1.0273x over previous
import functools

import jax
import jax.numpy as jnp
from jax import lax
from jax.experimental import pallas as pl
from jax.experimental.pallas import tpu as pltpu
from jax.experimental.pallas import tpu_sc as plsc

_EMBED = 32
_CODES = 8192
_BLOCK_M = 1024


_LANES = 128


def _dist_argmin_body(xn_ref, wn_ref, x2_ref, w_ref, idx_ref):
    x2 = x2_ref[...]
    w = w_ref[...]
    mm2 = lax.dot_general(
        x2.astype(jnp.bfloat16), w.astype(jnp.bfloat16),
        (((1,), (1,)), ((), ())), preferred_element_type=jnp.float32
    )
    xn = xn_ref[...]
    wn = wn_ref[...]
    nb = _CODES // 2
    ns = nb // _LANES
    lane = lax.broadcasted_iota(jnp.int32, (_BLOCK_M, _LANES), 1)
    acc_q = jnp.full((_BLOCK_M,), jnp.inf, dtype=jnp.float32)
    acc_i = jnp.zeros((_BLOCK_M,), dtype=jnp.int32)
    for b in range(2):
        base = b * nb
        run_v = (xn + wn[:, base:base + _LANES]) - mm2[:, base:base + _LANES]
        run_s = jnp.zeros((_BLOCK_M, _LANES), dtype=jnp.int32)
        for s in range(1, ns):
            lo = base + s * _LANES
            dv = (xn + wn[:, lo:lo + _LANES]) - mm2[:, lo:lo + _LANES]
            lt = dv < run_v
            run_v = jnp.where(lt, dv, run_v)
            run_s = jnp.where(lt, jnp.int32(s), run_s)
        m = jnp.min(run_v, axis=1)
        cand = run_s * jnp.int32(_LANES) + lane
        g = jnp.min(jnp.where(run_v == m[:, None], cand, jnp.int32(_CODES)), axis=1)
        take = m < acc_q
        acc_q = jnp.where(take, m.astype(jnp.bfloat16).astype(jnp.float32), acc_q)
        acc_i = jnp.where(take, g + jnp.int32(base), acc_i)
    idx_ref[...] = acc_i.reshape(1, 1, _BLOCK_M)


def _dist_argmin(xn, wn, flat, w, interpret=False):
    nb = flat.shape[0] // _BLOCK_M
    idx3 = pl.pallas_call(
        _dist_argmin_body,
        grid=(nb,),
        in_specs=[
            pl.BlockSpec((_BLOCK_M, 1), lambda i: (i, 0)),
            pl.BlockSpec((1, _CODES), lambda i: (0, 0)),
            pl.BlockSpec((_BLOCK_M, _EMBED), lambda i: (i, 0)),
            pl.BlockSpec((_CODES, _EMBED), lambda i: (0, 0)),
        ],
        out_specs=pl.BlockSpec((1, 1, _BLOCK_M), lambda i: (i, 0, 0)),
        out_shape=jax.ShapeDtypeStruct((nb, 1, _BLOCK_M), jnp.int32),
        interpret=interpret,
    )(xn, wn, flat, w)
    return idx3.reshape(-1)


_CHUNK = 128


def _sc_gather(table, idx, n_rows):
    info = plsc.get_sparse_core_info()
    nw = info.num_cores * info.num_subcores
    b_per_w = n_rows // nw
    n_ch = b_per_w // _CHUNK
    d = table.shape[1]
    mesh = plsc.VectorSubcoreMesh(core_axis_name="c", subcore_axis_name="s")

    @functools.partial(
        pl.kernel,
        mesh=mesh,
        out_type=jax.ShapeDtypeStruct((n_rows, d), jnp.float32),
        scratch_types=[
            pltpu.VMEM((n_ch, _CHUNK), jnp.int32),
            pltpu.VMEM((b_per_w, d), jnp.float32),
            pltpu.SemaphoreType.DMA,
        ],
        compiler_params=pltpu.CompilerParams(use_tc_tiling_on_sc=False),
    )
    def gather_k(table_hbm, idx_hbm, out_hbm, idx_v, rows_v, sem):
        wid = lax.axis_index("s") * info.num_cores + lax.axis_index("c")
        base = wid * b_per_w
        pltpu.sync_copy(idx_hbm.at[pl.ds(wid * n_ch, n_ch)], idx_v)
        copies = [
            pltpu.async_copy(
                table_hbm.at[idx_v.at[j]],
                rows_v.at[pl.ds(j * _CHUNK, _CHUNK)],
                sem,
            )
            for j in range(n_ch)
        ]
        for c in copies:
            c.wait()
        pltpu.sync_copy(rows_v, out_hbm.at[pl.ds(base, b_per_w)])

    return gather_k(table, idx.reshape(-1, _CHUNK))


def kernel(inputs, W):
    input_shape = inputs.shape
    flat = inputs.reshape(-1, _EMBED)
    n_rows = flat.shape[0]
    xn = jnp.sum(flat ** 2, axis=1, keepdims=True)
    wn = jnp.sum(W ** 2, axis=1).reshape(1, _CODES)
    idx = _dist_argmin(xn, wn, flat * 2.0, W)
    quantized = _sc_gather(W, idx, n_rows).reshape(input_shape)
    quantized_st = inputs + lax.stop_gradient(quantized - inputs)
    indices = idx.reshape(input_shape[:-1])
    return (quantized_st, indices, jnp.array(0.0, dtype=jnp.float32))
